# baseline (speedup 1.0000x reference)
import jax
import jax.numpy as jnp
from jax import lax
from jax.experimental import pallas as pl
from jax.experimental.pallas import tpu as pltpu

ROWS = 128
COLS = 32768
BLK = 16384
NBLK = COLS // BLK


def _body(x_ref, idx_ref, val_ref, m_scr, i_scr):
    k = pl.program_id(0)
    v = x_ref[...]
    bm = jnp.max(v, axis=1, keepdims=True)
    bi = jnp.argmax(v, axis=1, keepdims=True).astype(jnp.int32) + k * BLK

    @pl.when(k == 0)
    def _init():
        m_scr[...] = bm
        i_scr[...] = bi

    @pl.when(k != 0)
    def _acc():
        upd = bm > m_scr[...]
        m_scr[...] = jnp.where(upd, bm, m_scr[...])
        i_scr[...] = jnp.where(upd, bi, i_scr[...])

    @pl.when(k == NBLK - 1)
    def _out():
        idx_ref[...] = i_scr[...].reshape(ROWS)
        val_ref[...] = m_scr[...].reshape(ROWS)


def kernel(i):
    idx, vals = pl.pallas_call(
        _body,
        grid=(NBLK,),
        in_specs=[pl.BlockSpec((ROWS, BLK), lambda k: (0, k))],
        out_specs=[
            pl.BlockSpec((ROWS,), lambda k: (0,)),
            pl.BlockSpec((ROWS,), lambda k: (0,)),
        ],
        out_shape=[
            jax.ShapeDtypeStruct((ROWS,), jnp.int32),
            jax.ShapeDtypeStruct((ROWS,), jnp.float32),
        ],
        scratch_shapes=[
            pltpu.VMEM((ROWS, 1), jnp.float32),
            pltpu.VMEM((ROWS, 1), jnp.int32),
        ],
        compiler_params=pltpu.CompilerParams(
            dimension_semantics=("arbitrary",)
        ),
    )(i)
    return (idx, vals, idx)


# speedup vs baseline: 1.0256x; 1.0256x over previous
import jax
import jax.numpy as jnp
from jax import lax
from jax.experimental import pallas as pl
from jax.experimental.pallas import tpu as pltpu

ROWS = 128
COLS = 32768
BLKA = 14336
BLKB = 2048
NA = 2
NB = 2
GRID = NA + NB


def _fold(bm, bi, k, m_scr, i_scr):
    @pl.when(k == 0)
    def _init():
        m_scr[...] = bm
        i_scr[...] = bi

    @pl.when(k != 0)
    def _acc():
        upd = bm > m_scr[...]
        m_scr[...] = jnp.where(upd, bm, m_scr[...])
        i_scr[...] = jnp.where(upd, bi, i_scr[...])


def _body(a_ref, b_ref, idx_ref, val_ref, m_scr, i_scr):
    k = pl.program_id(0)

    @pl.when(k < NA)
    def _wide():
        v = a_ref[...]
        bm = jnp.max(v, axis=1, keepdims=True)
        iota = lax.broadcasted_iota(jnp.int32, (ROWS, BLKA), 1)
        bi = (
            jnp.min(jnp.where(v == bm, iota, COLS), axis=1, keepdims=True)
            + k * BLKA
        )
        _fold(bm, bi, k, m_scr, i_scr)

    @pl.when(k >= NA)
    def _narrow():
        v = b_ref[...]
        bm = jnp.max(v, axis=1, keepdims=True)
        iota = lax.broadcasted_iota(jnp.int32, (ROWS, BLKB), 1)
        bi = (
            jnp.min(jnp.where(v == bm, iota, COLS), axis=1, keepdims=True)
            + NA * BLKA
            + (k - NA) * BLKB
        )
        _fold(bm, bi, k, m_scr, i_scr)

    @pl.when(k == GRID - 1)
    def _out():
        idx_ref[...] = i_scr[...].reshape(ROWS)
        val_ref[...] = m_scr[...].reshape(ROWS)


def kernel(i):
    first_b = NA * BLKA // BLKB
    idx, vals = pl.pallas_call(
        _body,
        grid=(GRID,),
        in_specs=[
            pl.BlockSpec(
                (ROWS, BLKA), lambda k: (0, jnp.minimum(k, NA - 1))
            ),
            pl.BlockSpec(
                (ROWS, BLKB),
                lambda k: (0, jnp.maximum(k, NA) - NA + first_b),
            ),
        ],
        out_specs=[
            pl.BlockSpec((ROWS,), lambda k: (0,)),
            pl.BlockSpec((ROWS,), lambda k: (0,)),
        ],
        out_shape=[
            jax.ShapeDtypeStruct((ROWS,), jnp.int32),
            jax.ShapeDtypeStruct((ROWS,), jnp.float32),
        ],
        scratch_shapes=[
            pltpu.VMEM((ROWS, 1), jnp.float32),
            pltpu.VMEM((ROWS, 1), jnp.int32),
        ],
        compiler_params=pltpu.CompilerParams(
            dimension_semantics=("arbitrary",)
        ),
    )(i, i)
    return (idx, vals, idx)


# speedup vs baseline: 1.1448x; 1.1162x over previous
import jax
import jax.numpy as jnp
from jax import lax
from jax.experimental import pallas as pl
from jax.experimental.pallas import tpu as pltpu

ROWS = 128
COLS = 32768
BLK = 16384
NBLK = COLS // BLK


def _body(x_ref, idx_ref, val_ref, m_scr, i_scr):
    k = pl.program_id(0)
    v = x_ref[...]
    bm = jnp.max(v, axis=1, keepdims=True)
    iota = lax.broadcasted_iota(jnp.int32, (ROWS, BLK), 1)
    bi = jnp.min(jnp.where(v == bm, iota, COLS), axis=1, keepdims=True) + k * BLK

    @pl.when(k == 0)
    def _init():
        m_scr[...] = bm
        i_scr[...] = bi

    @pl.when(k != 0)
    def _acc():
        upd = bm > m_scr[...]
        m_scr[...] = jnp.where(upd, bm, m_scr[...])
        i_scr[...] = jnp.where(upd, bi, i_scr[...])

    @pl.when(k == NBLK - 1)
    def _out():
        idx_ref[...] = i_scr[...].reshape(ROWS)
        val_ref[...] = m_scr[...].reshape(ROWS)


def kernel(i):
    idx, vals = pl.pallas_call(
        _body,
        grid=(NBLK,),
        in_specs=[pl.BlockSpec((ROWS, BLK), lambda k: (0, k))],
        out_specs=[
            pl.BlockSpec((ROWS,), lambda k: (0,)),
            pl.BlockSpec((ROWS,), lambda k: (0,)),
        ],
        out_shape=[
            jax.ShapeDtypeStruct((ROWS,), jnp.int32),
            jax.ShapeDtypeStruct((ROWS,), jnp.float32),
        ],
        scratch_shapes=[
            pltpu.VMEM((ROWS, 1), jnp.float32),
            pltpu.VMEM((ROWS, 1), jnp.int32),
        ],
        compiler_params=pltpu.CompilerParams(
            dimension_semantics=("arbitrary",)
        ),
    )(i)
    return (idx, vals, idx)


# speedup vs baseline: 1.2899x; 1.1267x over previous
import jax
import jax.numpy as jnp
from jax import lax
from jax.experimental import pallas as pl
from jax.experimental.pallas import tpu as pltpu

ROWS = 128
COLS = 32768
BLK = 16384
NBLK = COLS // BLK


def _body(x_ref, idx_ref, val_ref, idx2_ref, m_scr, i_scr):
    k = pl.program_id(0)
    v = x_ref[...]
    bm = jnp.max(v, axis=1, keepdims=True)
    iota = lax.broadcasted_iota(jnp.int32, (ROWS, BLK), 1)
    bi = jnp.min(jnp.where(v == bm, iota, COLS), axis=1, keepdims=True) + k * BLK

    @pl.when(k == 0)
    def _init():
        m_scr[...] = bm
        i_scr[...] = bi

    @pl.when(k != 0)
    def _acc():
        upd = bm > m_scr[...]
        m_scr[...] = jnp.where(upd, bm, m_scr[...])
        i_scr[...] = jnp.where(upd, bi, i_scr[...])

    @pl.when(k == NBLK - 1)
    def _out():
        fi = i_scr[...].reshape(ROWS)
        idx_ref[...] = fi
        val_ref[...] = m_scr[...].reshape(ROWS)
        idx2_ref[...] = fi


def kernel(i):
    idx, vals, idx2 = pl.pallas_call(
        _body,
        grid=(NBLK,),
        in_specs=[pl.BlockSpec((ROWS, BLK), lambda k: (0, k))],
        out_specs=[
            pl.BlockSpec((ROWS,), lambda k: (0,)),
            pl.BlockSpec((ROWS,), lambda k: (0,)),
            pl.BlockSpec((ROWS,), lambda k: (0,)),
        ],
        out_shape=[
            jax.ShapeDtypeStruct((ROWS,), jnp.int32),
            jax.ShapeDtypeStruct((ROWS,), jnp.float32),
            jax.ShapeDtypeStruct((ROWS,), jnp.int32),
        ],
        scratch_shapes=[
            pltpu.VMEM((ROWS, 1), jnp.float32),
            pltpu.VMEM((ROWS, 1), jnp.int32),
        ],
        compiler_params=pltpu.CompilerParams(
            dimension_semantics=("arbitrary",)
        ),
    )(i)
    return (idx, vals, idx2)
